# baseline (speedup 1.0000x reference)
import functools

import jax
import jax.numpy as jnp
from jax.experimental import pallas as pl
from jax.experimental.pallas import tpu as pltpu
from jax.sharding import Mesh, PartitionSpec as P

_G = 32
_CHUNK = 2048
_NS_ITERS = 10


def _cov_kernel(x_ref, cov_ref, mean_ref, acc_ref, sum_ref, *, n_chunks, n_rows):
    i = pl.program_id(1)

    @pl.when(i == 0)
    def _():
        acc_ref[...] = jnp.zeros_like(acc_ref)
        sum_ref[...] = jnp.zeros_like(sum_ref)

    xb = x_ref[...]
    acc_ref[...] += jax.lax.dot_general(
        xb, xb, (((0,), (0,)), ((), ())), preferred_element_type=jnp.float32)
    sum_ref[...] += jnp.sum(xb, axis=0, keepdims=True)

    @pl.when(i == n_chunks - 1)
    def _():
        s = sum_ref[...]
        outer = jax.lax.dot_general(
            s, s, (((0,), (0,)), ((), ())), preferred_element_type=jnp.float32)
        cov_ref[0] = (acc_ref[...] - outer / n_rows) / (n_rows - 1)
        mean_ref[0] = jnp.broadcast_to(s / n_rows, mean_ref.shape[1:])


def _ns_kernel(cov_ref, mean_ref, w_ref, b_ref, *, n_iter):
    a = cov_ref[0]
    d = a.shape[0]
    rows = jax.lax.broadcasted_iota(jnp.int32, (d, d), 0)
    cols = jax.lax.broadcasted_iota(jnp.int32, (d, d), 1)
    eye = jnp.where(rows == cols, 1.0, 0.0).astype(jnp.float32)
    norm = jnp.max(jnp.sum(jnp.abs(a), axis=1, keepdims=True))
    y = a / norm
    z = eye
    for _ in range(n_iter):
        t = 1.5 * eye - 0.5 * jnp.dot(z, y, preferred_element_type=jnp.float32)
        y = jnp.dot(y, t, preferred_element_type=jnp.float32)
        z = jnp.dot(t, z, preferred_element_type=jnp.float32)
    w = z * jax.lax.rsqrt(norm)
    w_ref[0] = w
    mu = mean_ref[0, 0:1, :]
    b = jnp.dot(mu, w, preferred_element_type=jnp.float32)
    b_ref[0] = jnp.broadcast_to(b, b_ref.shape[1:])


def _apply_kernel(x_ref, w_ref, b_ref, o_ref):
    o_ref[...] = jnp.dot(
        x_ref[...], w_ref[0], preferred_element_type=jnp.float32) - b_ref[0, 0:1, :]


def _whiten(x, *, d, interpret):
    n, dim = x.shape
    g = dim // d
    n_chunks = n // _CHUNK

    cov, mean = pl.pallas_call(
        functools.partial(_cov_kernel, n_chunks=n_chunks, n_rows=n),
        grid=(g, n_chunks),
        in_specs=[pl.BlockSpec((_CHUNK, d), lambda gi, i: (i, gi))],
        out_specs=[
            pl.BlockSpec((1, d, d), lambda gi, i: (gi, 0, 0)),
            pl.BlockSpec((1, 8, d), lambda gi, i: (gi, 0, 0)),
        ],
        out_shape=[
            jax.ShapeDtypeStruct((g, d, d), jnp.float32),
            jax.ShapeDtypeStruct((g, 8, d), jnp.float32),
        ],
        scratch_shapes=[
            pltpu.VMEM((d, d), jnp.float32),
            pltpu.VMEM((1, d), jnp.float32),
        ],
        compiler_params=pltpu.CompilerParams(
            dimension_semantics=("parallel", "arbitrary")),
        name="group_cov",
        interpret=interpret,
    )(x)

    w, b = pl.pallas_call(
        functools.partial(_ns_kernel, n_iter=_NS_ITERS),
        grid=(g,),
        in_specs=[
            pl.BlockSpec((1, d, d), lambda gi: (gi, 0, 0)),
            pl.BlockSpec((1, 8, d), lambda gi: (gi, 0, 0)),
        ],
        out_specs=[
            pl.BlockSpec((1, d, d), lambda gi: (gi, 0, 0)),
            pl.BlockSpec((1, 8, d), lambda gi: (gi, 0, 0)),
        ],
        out_shape=[
            jax.ShapeDtypeStruct((g, d, d), jnp.float32),
            jax.ShapeDtypeStruct((g, 8, d), jnp.float32),
        ],
        compiler_params=pltpu.CompilerParams(
            dimension_semantics=("parallel",)),
        name="newton_schulz",
        interpret=interpret,
    )(cov, mean)

    out = pl.pallas_call(
        _apply_kernel,
        grid=(g, n_chunks),
        in_specs=[
            pl.BlockSpec((_CHUNK, d), lambda gi, i: (i, gi)),
            pl.BlockSpec((1, d, d), lambda gi, i: (gi, 0, 0)),
            pl.BlockSpec((1, 8, d), lambda gi, i: (gi, 0, 0)),
        ],
        out_specs=pl.BlockSpec((_CHUNK, d), lambda gi, i: (i, gi)),
        out_shape=jax.ShapeDtypeStruct((n, dim), jnp.float32),
        compiler_params=pltpu.CompilerParams(
            dimension_semantics=("parallel", "arbitrary")),
        name="whiten_apply",
        interpret=interpret,
    )(x, w, b)
    return out


@functools.partial(jax.jit, static_argnames=("interpret",))
def kernel(x, interpret=False):
    n, dim = x.shape
    d = dim // _G
    fn = functools.partial(_whiten, d=d, interpret=interpret)
    devs = jax.devices()
    n_shards = 2 if (len(devs) >= 2 and _G % 2 == 0) else 1
    if n_shards == 1:
        return fn(x)
    mesh = Mesh(devs[:n_shards], ("c",))
    return jax.shard_map(
        fn, mesh=mesh, in_specs=P(None, "c"), out_specs=P(None, "c"),
        check_vma=False)(x)


# speedup vs baseline: 1.5386x; 1.5386x over previous
import functools

import jax
import jax.numpy as jnp
from jax.experimental import pallas as pl
from jax.experimental.pallas import tpu as pltpu

_G = 32
_CHUNK = 2048
_NS_ITERS = 10


def _cov_kernel(x_ref, cov_ref, mean_ref, acc_ref, sum_ref, *, n_chunks, n_rows):
    i = pl.program_id(1)

    @pl.when(i == 0)
    def _():
        acc_ref[...] = jnp.zeros_like(acc_ref)
        sum_ref[...] = jnp.zeros_like(sum_ref)

    xb = x_ref[...]
    acc_ref[...] += jax.lax.dot_general(
        xb, xb, (((0,), (0,)), ((), ())), preferred_element_type=jnp.float32)
    sum_ref[...] += jnp.sum(xb, axis=0, keepdims=True)

    @pl.when(i == n_chunks - 1)
    def _():
        s = sum_ref[...]
        outer = jax.lax.dot_general(
            s, s, (((0,), (0,)), ((), ())), preferred_element_type=jnp.float32)
        cov_ref[0] = (acc_ref[...] - outer / n_rows) / (n_rows - 1)
        mean_ref[0] = jnp.broadcast_to(s / n_rows, mean_ref.shape[1:])


def _bmm(a, b):
    return jax.lax.dot_general(
        a, b, (((2,), (1,)), ((0,), (0,))), preferred_element_type=jnp.float32)


def _ns_kernel(cov_ref, mean_ref, w_ref, b_ref, *, n_iter):
    a = cov_ref[...]
    g, d, _ = a.shape
    rows = jax.lax.broadcasted_iota(jnp.int32, (g, d, d), 1)
    cols = jax.lax.broadcasted_iota(jnp.int32, (g, d, d), 2)
    eye = jnp.where(rows == cols, 1.0, 0.0).astype(jnp.float32)
    norm = jnp.max(jnp.sum(jnp.abs(a), axis=2, keepdims=True), axis=1,
                   keepdims=True)
    y = a / norm
    z = eye
    for _ in range(n_iter):
        t = 1.5 * eye - 0.5 * _bmm(z, y)
        y = _bmm(y, t)
        z = _bmm(t, z)
    w = z * jax.lax.rsqrt(norm)
    w_ref[...] = w
    mu = mean_ref[:, 0:1, :]
    b = jax.lax.dot_general(
        mu, w, (((2,), (1,)), ((0,), (0,))), preferred_element_type=jnp.float32)
    b_ref[...] = jnp.broadcast_to(b, b_ref.shape)


def _apply_kernel(x_ref, w_ref, b_ref, o_ref):
    o_ref[...] = jnp.dot(
        x_ref[...], w_ref[0], preferred_element_type=jnp.float32) - b_ref[0, 0:1, :]


@functools.partial(jax.jit, static_argnames=("interpret",))
def kernel(x, interpret=False):
    n, dim = x.shape
    d = dim // _G
    n_chunks = n // _CHUNK

    cov, mean = pl.pallas_call(
        functools.partial(_cov_kernel, n_chunks=n_chunks, n_rows=n),
        grid=(_G, n_chunks),
        in_specs=[pl.BlockSpec((_CHUNK, d), lambda g, i: (i, g))],
        out_specs=[
            pl.BlockSpec((1, d, d), lambda g, i: (g, 0, 0)),
            pl.BlockSpec((1, 8, d), lambda g, i: (g, 0, 0)),
        ],
        out_shape=[
            jax.ShapeDtypeStruct((_G, d, d), jnp.float32),
            jax.ShapeDtypeStruct((_G, 8, d), jnp.float32),
        ],
        scratch_shapes=[
            pltpu.VMEM((d, d), jnp.float32),
            pltpu.VMEM((1, d), jnp.float32),
        ],
        compiler_params=pltpu.CompilerParams(
            dimension_semantics=("parallel", "arbitrary")),
        name="group_cov",
        interpret=interpret,
    )(x)

    w, b = pl.pallas_call(
        functools.partial(_ns_kernel, n_iter=_NS_ITERS),
        out_shape=[
            jax.ShapeDtypeStruct((_G, d, d), jnp.float32),
            jax.ShapeDtypeStruct((_G, 8, d), jnp.float32),
        ],
        compiler_params=pltpu.CompilerParams(
            vmem_limit_bytes=56 * 1024 * 1024),
        name="newton_schulz",
        interpret=interpret,
    )(cov, mean)

    out = pl.pallas_call(
        _apply_kernel,
        grid=(_G, n_chunks),
        in_specs=[
            pl.BlockSpec((_CHUNK, d), lambda g, i: (i, g)),
            pl.BlockSpec((1, d, d), lambda g, i: (g, 0, 0)),
            pl.BlockSpec((1, 8, d), lambda g, i: (g, 0, 0)),
        ],
        out_specs=pl.BlockSpec((_CHUNK, d), lambda g, i: (i, g)),
        out_shape=jax.ShapeDtypeStruct((n, dim), jnp.float32),
        compiler_params=pltpu.CompilerParams(
            dimension_semantics=("parallel", "arbitrary")),
        name="whiten_apply",
        interpret=interpret,
    )(x, w, b)
    return out


# speedup vs baseline: 2.1459x; 1.3947x over previous
import functools

import jax
import jax.numpy as jnp
from jax.experimental import pallas as pl
from jax.experimental.pallas import tpu as pltpu

_G = 32
_CHUNK = 4096
_NS_ITERS = 10


def _cov_kernel(x_ref, cov_ref, mean_ref, acc_ref, sum_ref, *, n_chunks, n_rows):
    i = pl.program_id(1)

    @pl.when(i == 0)
    def _():
        acc_ref[...] = jnp.zeros_like(acc_ref)
        sum_ref[...] = jnp.zeros_like(sum_ref)

    xb = x_ref[...]
    acc_ref[...] += jax.lax.dot_general(
        xb, xb, (((0,), (0,)), ((), ())), preferred_element_type=jnp.float32)
    sum_ref[...] += jnp.sum(xb, axis=0, keepdims=True)

    @pl.when(i == n_chunks - 1)
    def _():
        s = sum_ref[...]
        outer = jax.lax.dot_general(
            s, s, (((0,), (0,)), ((), ())), preferred_element_type=jnp.float32)
        cov_ref[0] = (acc_ref[...] - outer / n_rows) / (n_rows - 1)
        mean_ref[0] = jnp.broadcast_to(s / n_rows, mean_ref.shape[1:])


def _bmm(a, b):
    return jax.lax.dot_general(
        a, b, (((2,), (1,)), ((0,), (0,))), preferred_element_type=jnp.float32)


def _ns_kernel(cov_ref, mean_ref, w_ref, b_ref, *, n_iter):
    a = cov_ref[...]
    g, d, _ = a.shape
    rows = jax.lax.broadcasted_iota(jnp.int32, (g, d, d), 1)
    cols = jax.lax.broadcasted_iota(jnp.int32, (g, d, d), 2)
    eye = jnp.where(rows == cols, 1.0, 0.0).astype(jnp.float32)
    norm = jnp.max(jnp.sum(jnp.abs(a), axis=2, keepdims=True), axis=1,
                   keepdims=True)
    y = a / norm
    z = eye
    for _ in range(n_iter):
        t = 1.5 * eye - 0.5 * _bmm(z, y)
        y = _bmm(y, t)
        z = _bmm(t, z)
    w = z * jax.lax.rsqrt(norm)
    w_ref[...] = w
    mu = mean_ref[:, 0:1, :]
    b = jax.lax.dot_general(
        mu, w, (((2,), (1,)), ((0,), (0,))), preferred_element_type=jnp.float32)
    b_ref[...] = jnp.broadcast_to(b, b_ref.shape)


def _apply_kernel(x_ref, w_ref, b_ref, o_ref):
    o_ref[...] = jnp.dot(
        x_ref[...], w_ref[0], preferred_element_type=jnp.float32) - b_ref[0, 0:1, :]


@functools.partial(jax.jit, static_argnames=("interpret",))
def kernel(x, interpret=False):
    n, dim = x.shape
    d = dim // _G
    n_chunks = n // _CHUNK

    cov, mean = pl.pallas_call(
        functools.partial(_cov_kernel, n_chunks=n_chunks, n_rows=n),
        grid=(_G, n_chunks),
        in_specs=[pl.BlockSpec((_CHUNK, d), lambda g, i: (i, g))],
        out_specs=[
            pl.BlockSpec((1, d, d), lambda g, i: (g, 0, 0)),
            pl.BlockSpec((1, 8, d), lambda g, i: (g, 0, 0)),
        ],
        out_shape=[
            jax.ShapeDtypeStruct((_G, d, d), jnp.float32),
            jax.ShapeDtypeStruct((_G, 8, d), jnp.float32),
        ],
        scratch_shapes=[
            pltpu.VMEM((d, d), jnp.float32),
            pltpu.VMEM((1, d), jnp.float32),
        ],
        compiler_params=pltpu.CompilerParams(
            dimension_semantics=("parallel", "arbitrary")),
        name="group_cov",
        interpret=interpret,
    )(x)

    w, b = pl.pallas_call(
        functools.partial(_ns_kernel, n_iter=_NS_ITERS),
        out_shape=[
            jax.ShapeDtypeStruct((_G, d, d), jnp.float32),
            jax.ShapeDtypeStruct((_G, 8, d), jnp.float32),
        ],
        compiler_params=pltpu.CompilerParams(
            vmem_limit_bytes=56 * 1024 * 1024),
        name="newton_schulz",
        interpret=interpret,
    )(cov, mean)

    out = pl.pallas_call(
        _apply_kernel,
        grid=(_G, n_chunks),
        in_specs=[
            pl.BlockSpec((_CHUNK, d), lambda g, i: (i, g)),
            pl.BlockSpec((1, d, d), lambda g, i: (g, 0, 0)),
            pl.BlockSpec((1, 8, d), lambda g, i: (g, 0, 0)),
        ],
        out_specs=pl.BlockSpec((_CHUNK, d), lambda g, i: (i, g)),
        out_shape=jax.ShapeDtypeStruct((n, dim), jnp.float32),
        compiler_params=pltpu.CompilerParams(
            dimension_semantics=("parallel", "arbitrary")),
        name="whiten_apply",
        interpret=interpret,
    )(x, w, b)
    return out


# speedup vs baseline: 2.5637x; 1.1947x over previous
import functools

import jax
import jax.numpy as jnp
from jax.experimental import pallas as pl
from jax.experimental.pallas import tpu as pltpu

_G = 32
_CHUNK = 8192
_NS_ITERS = 10


def _cov_kernel(x_ref, cov_ref, mean_ref, acc_ref, sum_ref, *, n_chunks, n_rows):
    i = pl.program_id(1)

    @pl.when(i == 0)
    def _():
        acc_ref[...] = jnp.zeros_like(acc_ref)
        sum_ref[...] = jnp.zeros_like(sum_ref)

    xb = x_ref[...]
    acc_ref[...] += jax.lax.dot_general(
        xb, xb, (((0,), (0,)), ((), ())), preferred_element_type=jnp.float32)
    sum_ref[...] += jnp.sum(xb, axis=0, keepdims=True)

    @pl.when(i == n_chunks - 1)
    def _():
        s = sum_ref[...]
        outer = jax.lax.dot_general(
            s, s, (((0,), (0,)), ((), ())), preferred_element_type=jnp.float32)
        cov_ref[0] = (acc_ref[...] - outer / n_rows) / (n_rows - 1)
        mean_ref[0] = jnp.broadcast_to(s / n_rows, mean_ref.shape[1:])


def _bmm(a, b):
    return jax.lax.dot_general(
        a, b, (((2,), (1,)), ((0,), (0,))), preferred_element_type=jnp.float32)


def _ns_kernel(cov_ref, mean_ref, w_ref, b_ref, *, n_iter):
    a = cov_ref[...]
    g, d, _ = a.shape
    rows = jax.lax.broadcasted_iota(jnp.int32, (g, d, d), 1)
    cols = jax.lax.broadcasted_iota(jnp.int32, (g, d, d), 2)
    eye = jnp.where(rows == cols, 1.0, 0.0).astype(jnp.float32)
    norm = jnp.max(jnp.sum(jnp.abs(a), axis=2, keepdims=True), axis=1,
                   keepdims=True)
    y = a / norm
    z = eye
    for _ in range(n_iter):
        t = 1.5 * eye - 0.5 * _bmm(z, y)
        y = _bmm(y, t)
        z = _bmm(t, z)
    w = z * jax.lax.rsqrt(norm)
    w_ref[...] = w
    mu = mean_ref[:, 0:1, :]
    b = jax.lax.dot_general(
        mu, w, (((2,), (1,)), ((0,), (0,))), preferred_element_type=jnp.float32)
    b_ref[...] = jnp.broadcast_to(b, b_ref.shape)


def _apply_kernel(x_ref, w_ref, b_ref, o_ref):
    o_ref[...] = jnp.dot(
        x_ref[...], w_ref[0], preferred_element_type=jnp.float32) - b_ref[0, 0:1, :]


@functools.partial(jax.jit, static_argnames=("interpret",))
def kernel(x, interpret=False):
    n, dim = x.shape
    d = dim // _G
    n_chunks = n // _CHUNK

    cov, mean = pl.pallas_call(
        functools.partial(_cov_kernel, n_chunks=n_chunks, n_rows=n),
        grid=(_G, n_chunks),
        in_specs=[pl.BlockSpec((_CHUNK, d), lambda g, i: (i, g))],
        out_specs=[
            pl.BlockSpec((1, d, d), lambda g, i: (g, 0, 0)),
            pl.BlockSpec((1, 8, d), lambda g, i: (g, 0, 0)),
        ],
        out_shape=[
            jax.ShapeDtypeStruct((_G, d, d), jnp.float32),
            jax.ShapeDtypeStruct((_G, 8, d), jnp.float32),
        ],
        scratch_shapes=[
            pltpu.VMEM((d, d), jnp.float32),
            pltpu.VMEM((1, d), jnp.float32),
        ],
        compiler_params=pltpu.CompilerParams(
            dimension_semantics=("parallel", "arbitrary")),
        name="group_cov",
        interpret=interpret,
    )(x)

    w, b = pl.pallas_call(
        functools.partial(_ns_kernel, n_iter=_NS_ITERS),
        out_shape=[
            jax.ShapeDtypeStruct((_G, d, d), jnp.float32),
            jax.ShapeDtypeStruct((_G, 8, d), jnp.float32),
        ],
        compiler_params=pltpu.CompilerParams(
            vmem_limit_bytes=56 * 1024 * 1024),
        name="newton_schulz",
        interpret=interpret,
    )(cov, mean)

    out = pl.pallas_call(
        _apply_kernel,
        grid=(_G, n_chunks),
        in_specs=[
            pl.BlockSpec((_CHUNK, d), lambda g, i: (i, g)),
            pl.BlockSpec((1, d, d), lambda g, i: (g, 0, 0)),
            pl.BlockSpec((1, 8, d), lambda g, i: (g, 0, 0)),
        ],
        out_specs=pl.BlockSpec((_CHUNK, d), lambda g, i: (i, g)),
        out_shape=jax.ShapeDtypeStruct((n, dim), jnp.float32),
        compiler_params=pltpu.CompilerParams(
            dimension_semantics=("parallel", "arbitrary")),
        name="whiten_apply",
        interpret=interpret,
    )(x, w, b)
    return out


# speedup vs baseline: 2.8302x; 1.1039x over previous
import functools

import jax
import jax.numpy as jnp
from jax.experimental import pallas as pl
from jax.experimental.pallas import tpu as pltpu

_G = 32
_CHUNK = 16384
_NS_ITERS = 10


def _cov_kernel(x_ref, cov_ref, mean_ref, acc_ref, sum_ref, *, n_chunks, n_rows):
    i = pl.program_id(1)

    @pl.when(i == 0)
    def _():
        acc_ref[...] = jnp.zeros_like(acc_ref)
        sum_ref[...] = jnp.zeros_like(sum_ref)

    xb = x_ref[...]
    acc_ref[...] += jax.lax.dot_general(
        xb, xb, (((0,), (0,)), ((), ())), preferred_element_type=jnp.float32)
    sum_ref[...] += jnp.sum(xb, axis=0, keepdims=True)

    @pl.when(i == n_chunks - 1)
    def _():
        s = sum_ref[...]
        outer = jax.lax.dot_general(
            s, s, (((0,), (0,)), ((), ())), preferred_element_type=jnp.float32)
        cov_ref[0] = (acc_ref[...] - outer / n_rows) / (n_rows - 1)
        mean_ref[0] = jnp.broadcast_to(s / n_rows, mean_ref.shape[1:])


def _bmm(a, b):
    return jax.lax.dot_general(
        a, b, (((2,), (1,)), ((0,), (0,))), preferred_element_type=jnp.float32)


def _ns_kernel(cov_ref, mean_ref, w_ref, b_ref, *, n_iter):
    a = cov_ref[...]
    g, d, _ = a.shape
    rows = jax.lax.broadcasted_iota(jnp.int32, (g, d, d), 1)
    cols = jax.lax.broadcasted_iota(jnp.int32, (g, d, d), 2)
    eye = jnp.where(rows == cols, 1.0, 0.0).astype(jnp.float32)
    norm = jnp.max(jnp.sum(jnp.abs(a), axis=2, keepdims=True), axis=1,
                   keepdims=True)
    y = a / norm
    z = eye
    for _ in range(n_iter):
        t = 1.5 * eye - 0.5 * _bmm(z, y)
        y = _bmm(y, t)
        z = _bmm(t, z)
    w = z * jax.lax.rsqrt(norm)
    w_ref[...] = w
    mu = mean_ref[:, 0:1, :]
    b = jax.lax.dot_general(
        mu, w, (((2,), (1,)), ((0,), (0,))), preferred_element_type=jnp.float32)
    b_ref[...] = jnp.broadcast_to(b, b_ref.shape)


def _apply_kernel(x_ref, w_ref, b_ref, o_ref):
    o_ref[...] = jnp.dot(
        x_ref[...], w_ref[0], preferred_element_type=jnp.float32) - b_ref[0, 0:1, :]


@functools.partial(jax.jit, static_argnames=("interpret",))
def kernel(x, interpret=False):
    n, dim = x.shape
    d = dim // _G
    n_chunks = n // _CHUNK

    cov, mean = pl.pallas_call(
        functools.partial(_cov_kernel, n_chunks=n_chunks, n_rows=n),
        grid=(_G, n_chunks),
        in_specs=[pl.BlockSpec((_CHUNK, d), lambda g, i: (i, g))],
        out_specs=[
            pl.BlockSpec((1, d, d), lambda g, i: (g, 0, 0)),
            pl.BlockSpec((1, 8, d), lambda g, i: (g, 0, 0)),
        ],
        out_shape=[
            jax.ShapeDtypeStruct((_G, d, d), jnp.float32),
            jax.ShapeDtypeStruct((_G, 8, d), jnp.float32),
        ],
        scratch_shapes=[
            pltpu.VMEM((d, d), jnp.float32),
            pltpu.VMEM((1, d), jnp.float32),
        ],
        compiler_params=pltpu.CompilerParams(
            dimension_semantics=("parallel", "arbitrary"),
            vmem_limit_bytes=56 * 1024 * 1024),
        name="group_cov",
        interpret=interpret,
    )(x)

    w, b = pl.pallas_call(
        functools.partial(_ns_kernel, n_iter=_NS_ITERS),
        out_shape=[
            jax.ShapeDtypeStruct((_G, d, d), jnp.float32),
            jax.ShapeDtypeStruct((_G, 8, d), jnp.float32),
        ],
        compiler_params=pltpu.CompilerParams(
            vmem_limit_bytes=56 * 1024 * 1024),
        name="newton_schulz",
        interpret=interpret,
    )(cov, mean)

    out = pl.pallas_call(
        _apply_kernel,
        grid=(_G, n_chunks),
        in_specs=[
            pl.BlockSpec((_CHUNK, d), lambda g, i: (i, g)),
            pl.BlockSpec((1, d, d), lambda g, i: (g, 0, 0)),
            pl.BlockSpec((1, 8, d), lambda g, i: (g, 0, 0)),
        ],
        out_specs=pl.BlockSpec((_CHUNK, d), lambda g, i: (i, g)),
        out_shape=jax.ShapeDtypeStruct((n, dim), jnp.float32),
        compiler_params=pltpu.CompilerParams(
            dimension_semantics=("parallel", "arbitrary"),
            vmem_limit_bytes=56 * 1024 * 1024),
        name="whiten_apply",
        interpret=interpret,
    )(x, w, b)
    return out
